# baseline (speedup 1.0000x reference)
import jax
import jax.numpy as jnp
from jax.experimental import pallas as pl
from jax.experimental.pallas import tpu as pltpu

B = 2048
D = 1024
E = 8
K = 2
GI = D + 64 + 4

R_BLK = 256
BI = 512
BJ = 512

_INV_SQRT2 = 0.7071067811865476


def _dot_t(a, w):
    return jax.lax.dot_general(
        a, w, (((1,), (1,)), ((), ())), precision=jax.lax.Precision.DEFAULT,
        preferred_element_type=jnp.float32)


def _gelu(v):
    return 0.5 * v * (1.0 + jax.lax.erf(v * _INV_SQRT2))


def _mlp_kernel(x_ref, bw_ref, bb_ref, bg_ref, bbeta_ref, m_ref,
                rc1w_ref, rc1b_ref, rc2w_ref, rc2b_ref,
                g1w_ref, g1b_ref, g1g_ref, g1beta_ref,
                g2w_ref, g2b_ref, g2g_ref, g2beta_ref,
                g3w_ref, g3b_ref, ns1w_ref, ns1b_ref, ns2w_ref, ns2b_ref,
                exps_ref, clean_ref, ns_ref, ps_ref):
    i = pl.program_id(0)
    xb = x_ref[:, :]
    h = jnp.maximum(_dot_t(xb, bw_ref[:, :]) + bb_ref[:, :], 0.0)
    gm = jnp.dot(h, m_ref[:, :], precision=jax.lax.Precision.HIGHEST,
                 preferred_element_type=jnp.float32)
    gx2 = jnp.dot(h * h, m_ref[:, :], precision=jax.lax.Precision.HIGHEST,
                  preferred_element_type=jnp.float32)
    gv = gx2 - gm * gm
    mf = (h - gm) * jax.lax.rsqrt(gv + 1e-5) * bg_ref[:, :] + bbeta_ref[:, :]
    rc = jnp.maximum(_dot_t(mf, rc1w_ref[:, :]) + rc1b_ref[:, :], 0.0)
    rl = _dot_t(rc, rc2w_ref[:, :]) + rc2b_ref[:, :]
    rl = rl - jnp.max(rl, axis=1, keepdims=True)
    re_ = jnp.exp(rl)
    regime = re_ / jnp.sum(re_, axis=1, keepdims=True)
    gi = jnp.concatenate([xb, mf, regime], axis=1)
    h1 = _dot_t(gi, g1w_ref[:, :]) + g1b_ref[:, :]
    m1 = jnp.sum(h1, axis=1, keepdims=True) * (1.0 / (2 * GI))
    d1 = h1 - m1
    v1 = jnp.sum(d1 * d1, axis=1, keepdims=True) * (1.0 / (2 * GI))
    h1 = _gelu(d1 * jax.lax.rsqrt(v1 + 1e-5) * g1g_ref[:, :] + g1beta_ref[:, :])
    h2 = _dot_t(h1, g2w_ref[:, :]) + g2b_ref[:, :]
    m2 = jnp.sum(h2, axis=1, keepdims=True) * (1.0 / GI)
    d2 = h2 - m2
    v2 = jnp.sum(d2 * d2, axis=1, keepdims=True) * (1.0 / GI)
    h2 = _gelu(d2 * jax.lax.rsqrt(v2 + 1e-5) * g2g_ref[:, :] + g2beta_ref[:, :])
    clean = (_dot_t(h2, g3w_ref[:, :]) + g3b_ref[:, :]
             + _dot_t(regime, exps_ref[:, :]))
    nsh = jnp.maximum(_dot_t(regime, ns1w_ref[:, :]) + ns1b_ref[:, :], 0.0)
    nsb = nsh.astype(jnp.bfloat16).astype(jnp.float32)
    nwb = ns2w_ref[:, :].astype(jnp.bfloat16).astype(jnp.float32)
    nss = jax.nn.sigmoid(
        jnp.sum(nsb * nwb, axis=1, keepdims=True) + ns2b_ref[0, 0])
    clean_ref[:, :] = clean
    ns_ref[:, :] = nss
    cm = jnp.max(clean, axis=1, keepdims=True)
    ce = jnp.exp(clean - cm)
    gs = ce / jnp.sum(ce, axis=1, keepdims=True)
    psb = jnp.sum(gs, axis=0, keepdims=True)

    @pl.when(i == 0)
    def _():
        ps_ref[:, :] = psb

    @pl.when(i > 0)
    def _():
        ps_ref[:, :] = ps_ref[:, :] + psb


def _topk_kernel(ct_ref, nt_ref, ns_ref, t0_ref, t1_ref, i0_ref, i1_ref,
                 oh_ref):
    ib = pl.program_id(1)
    nsv = ns_ref[:, :]
    neg = jnp.full((BI, BJ), -jnp.inf, jnp.float32)
    m1, m2 = neg, neg
    mask = jnp.int32(~7)
    for e in range(E):
        a = ct_ref[e:e + 1, :] + nsv * nt_ref[e:e + 1, :]
        ai = jax.lax.bitcast_convert_type(
            (jax.lax.bitcast_convert_type(a, jnp.int32) & mask) | e,
            jnp.float32)
        t = jnp.minimum(m1, ai)
        m1 = jnp.maximum(m1, ai)
        m2 = jnp.maximum(m2, t)
    e1 = jax.lax.bitcast_convert_type(m1, jnp.int32) & 7
    e2 = jax.lax.bitcast_convert_type(m2, jnp.int32) & 7
    s0 = jax.nn.sigmoid(m1 - m2)
    t0_ref[:, :] = s0
    t1_ref[:, :] = 1.0 - s0
    i0_ref[:, :] = e1
    i1_ref[:, :] = e2
    msk = jnp.left_shift(jnp.int32(1), e1) | jnp.left_shift(jnp.int32(1), e2)
    r = BI
    while r > 1:
        r //= 2
        msk = msk[:r, :] | msk[r:2 * r, :]

    @pl.when(ib == 0)
    def _():
        oh_ref[:, :] = msk

    @pl.when(ib > 0)
    def _():
        oh_ref[:, :] = oh_ref[:, :] | msk


def _loss_kernel(ps_ref, oh_ref, out_ref):
    ps = ps_ref[:, :]
    ohm = oh_ref[:, :]
    acc = jnp.zeros((1, 1), jnp.float32)
    for e in range(E):
        cnt = jnp.sum(((ohm >> e) & 1).astype(jnp.float32))
        acc = acc + ps[0, e] * cnt
    load_loss = E * acc * (1.0 / (B * B))
    m = jnp.sum(ps) * (1.0 / E)
    v = jnp.sum((ps - m) ** 2) * (1.0 / (E - 1))
    out_ref[:, :] = 0.01 * load_loss + 0.01 * (v / m)


def kernel(x, params, noise_raw):
    p = params
    f32 = jnp.float32
    row = lambda a: a.reshape(1, -1).astype(f32)
    bw = jnp.concatenate([p[n + "_w"] for n in ("ve", "te", "me", "re")], 0)
    bb = jnp.concatenate([p[n + "_b"] for n in ("ve", "te", "me", "re")]).reshape(1, -1)
    bg = jnp.concatenate([p[n + "_g"] for n in ("ve", "te", "me", "re")]).reshape(1, -1)
    bbeta = jnp.concatenate([p[n + "_beta"] for n in ("ve", "te", "me", "re")]).reshape(1, -1)
    gmat = jnp.kron(jnp.eye(4, dtype=f32), jnp.full((16, 16), 1.0 / 16, f32))

    nrb = B // R_BLK
    full = lambda s: pl.BlockSpec(s, lambda i: (0, 0))
    clean, ns, ps = pl.pallas_call(
        _mlp_kernel,
        grid=(nrb,),
        in_specs=[
            pl.BlockSpec((R_BLK, D), lambda i: (i, 0)),
            full((64, D)), full((1, 64)), full((1, 64)), full((1, 64)),
            full((64, 64)),
            full((32, 64)), full((1, 32)), full((4, 32)), full((1, 4)),
            full((2 * GI, GI)), full((1, 2 * GI)), full((1, 2 * GI)),
            full((1, 2 * GI)),
            full((GI, 2 * GI)), full((1, GI)), full((1, GI)), full((1, GI)),
            full((E, GI)), full((1, E)),
            full((16, 4)), full((1, 16)), full((1, 16)), full((1, 1)),
            full((E, 4)),
        ],
        out_specs=[
            pl.BlockSpec((R_BLK, E), lambda i: (i, 0)),
            pl.BlockSpec((R_BLK, 1), lambda i: (i, 0)),
            pl.BlockSpec((1, E), lambda i: (0, 0)),
        ],
        out_shape=[
            jax.ShapeDtypeStruct((B, E), f32),
            jax.ShapeDtypeStruct((B, 1), f32),
            jax.ShapeDtypeStruct((1, E), f32),
        ],
    )(x, bw, bb, bg, bbeta, gmat,
      p["rc1_w"], row(p["rc1_b"]), p["rc2_w"], row(p["rc2_b"]),
      p["g1_w"], row(p["g1_b"]), row(p["g1_g"]), row(p["g1_beta"]),
      p["g2_w"], row(p["g2_b"]), row(p["g2_g"]), row(p["g2_beta"]),
      p["g3_w"], row(p["g3_b"]),
      p["ns1_w"], row(p["ns1_b"]), p["ns2_w"], row(p["ns2_b"]),
      p["exp_spec"])

    ct = clean.T
    nt = noise_raw.T
    t0, t1, i0, i1, oh = pl.pallas_call(
        _topk_kernel,
        grid=(B // BJ, B // BI),
        in_specs=[
            pl.BlockSpec((E, BJ), lambda j, i: (0, j)),
            pl.BlockSpec((E, BJ), lambda j, i: (0, j)),
            pl.BlockSpec((BI, 1), lambda j, i: (i, 0)),
        ],
        out_specs=[
            pl.BlockSpec((BI, BJ), lambda j, i: (i, j)),
            pl.BlockSpec((BI, BJ), lambda j, i: (i, j)),
            pl.BlockSpec((BI, BJ), lambda j, i: (i, j)),
            pl.BlockSpec((BI, BJ), lambda j, i: (i, j)),
            pl.BlockSpec((1, BJ), lambda j, i: (0, j)),
        ],
        out_shape=[
            jax.ShapeDtypeStruct((B, B), f32),
            jax.ShapeDtypeStruct((B, B), f32),
            jax.ShapeDtypeStruct((B, B), jnp.int32),
            jax.ShapeDtypeStruct((B, B), jnp.int32),
            jax.ShapeDtypeStruct((1, B), jnp.int32),
        ],
    )(ct, nt, ns)

    loss = pl.pallas_call(
        _loss_kernel,
        in_specs=[pl.BlockSpec((1, E), lambda: (0, 0)),
                  pl.BlockSpec((1, B), lambda: (0, 0))],
        out_specs=pl.BlockSpec((1, 1), lambda: (0, 0)),
        out_shape=jax.ShapeDtypeStruct((1, 1), f32),
    )(ps, oh)

    tw = jnp.stack([t0, t1], axis=-1)
    idx = jnp.stack([i0, i1], axis=-1)
    return (tw, idx, loss[0, 0])


# speedup vs baseline: 1.0034x; 1.0034x over previous
import jax
import jax.numpy as jnp
from jax.experimental import pallas as pl
from jax.experimental.pallas import tpu as pltpu

B = 2048
D = 1024
E = 8
K = 2
GI = D + 64 + 4

R_BLK = 256
BI = 128
BJ = 512

_INV_SQRT2 = 0.7071067811865476


def _dot_t(a, w):
    return jax.lax.dot_general(
        a, w, (((1,), (1,)), ((), ())), precision=jax.lax.Precision.DEFAULT,
        preferred_element_type=jnp.float32)


def _gelu(v):
    return 0.5 * v * (1.0 + jax.lax.erf(v * _INV_SQRT2))


def _mlp_kernel(x_ref, bw_ref, bb_ref, bg_ref, bbeta_ref, m_ref,
                rc1w_ref, rc1b_ref, rc2w_ref, rc2b_ref,
                g1w_ref, g1b_ref, g1g_ref, g1beta_ref,
                g2w_ref, g2b_ref, g2g_ref, g2beta_ref,
                g3w_ref, g3b_ref, ns1w_ref, ns1b_ref, ns2w_ref, ns2b_ref,
                exps_ref, clean_ref, ns_ref, ps_ref):
    i = pl.program_id(0)
    xb = x_ref[:, :]
    h = jnp.maximum(_dot_t(xb, bw_ref[:, :]) + bb_ref[:, :], 0.0)
    gm = jnp.dot(h, m_ref[:, :], precision=jax.lax.Precision.HIGHEST,
                 preferred_element_type=jnp.float32)
    gx2 = jnp.dot(h * h, m_ref[:, :], precision=jax.lax.Precision.HIGHEST,
                  preferred_element_type=jnp.float32)
    gv = gx2 - gm * gm
    mf = (h - gm) * jax.lax.rsqrt(gv + 1e-5) * bg_ref[:, :] + bbeta_ref[:, :]
    rc = jnp.maximum(_dot_t(mf, rc1w_ref[:, :]) + rc1b_ref[:, :], 0.0)
    rl = _dot_t(rc, rc2w_ref[:, :]) + rc2b_ref[:, :]
    rl = rl - jnp.max(rl, axis=1, keepdims=True)
    re_ = jnp.exp(rl)
    regime = re_ / jnp.sum(re_, axis=1, keepdims=True)
    gi = jnp.concatenate([xb, mf, regime], axis=1)
    h1 = _dot_t(gi, g1w_ref[:, :]) + g1b_ref[:, :]
    m1 = jnp.sum(h1, axis=1, keepdims=True) * (1.0 / (2 * GI))
    d1 = h1 - m1
    v1 = jnp.sum(d1 * d1, axis=1, keepdims=True) * (1.0 / (2 * GI))
    h1 = _gelu(d1 * jax.lax.rsqrt(v1 + 1e-5) * g1g_ref[:, :] + g1beta_ref[:, :])
    h2 = _dot_t(h1, g2w_ref[:, :]) + g2b_ref[:, :]
    m2 = jnp.sum(h2, axis=1, keepdims=True) * (1.0 / GI)
    d2 = h2 - m2
    v2 = jnp.sum(d2 * d2, axis=1, keepdims=True) * (1.0 / GI)
    h2 = _gelu(d2 * jax.lax.rsqrt(v2 + 1e-5) * g2g_ref[:, :] + g2beta_ref[:, :])
    clean = (_dot_t(h2, g3w_ref[:, :]) + g3b_ref[:, :]
             + _dot_t(regime, exps_ref[:, :]))
    nsh = jnp.maximum(_dot_t(regime, ns1w_ref[:, :]) + ns1b_ref[:, :], 0.0)
    nsb = nsh.astype(jnp.bfloat16).astype(jnp.float32)
    nwb = ns2w_ref[:, :].astype(jnp.bfloat16).astype(jnp.float32)
    nss = jax.nn.sigmoid(
        jnp.sum(nsb * nwb, axis=1, keepdims=True) + ns2b_ref[0, 0])
    clean_ref[:, :] = clean
    ns_ref[:, :] = nss
    cm = jnp.max(clean, axis=1, keepdims=True)
    ce = jnp.exp(clean - cm)
    gs = ce / jnp.sum(ce, axis=1, keepdims=True)
    psb = jnp.sum(gs, axis=0, keepdims=True)

    @pl.when(i == 0)
    def _():
        ps_ref[:, :] = psb

    @pl.when(i > 0)
    def _():
        ps_ref[:, :] = ps_ref[:, :] + psb


def _topk_kernel(ct_ref, nt_ref, ns_ref, t0_ref, t1_ref, i0_ref, i1_ref,
                 oh_ref):
    ib = pl.program_id(1)
    nsv = ns_ref[:, :]
    neg = jnp.full((BI, BJ), -jnp.inf, jnp.float32)
    m1, m2 = neg, neg
    mask = jnp.int32(~7)
    for e in range(E):
        a = ct_ref[e:e + 1, :] + nsv * nt_ref[e:e + 1, :]
        ai = jax.lax.bitcast_convert_type(
            (jax.lax.bitcast_convert_type(a, jnp.int32) & mask) | e,
            jnp.float32)
        t = jnp.minimum(m1, ai)
        m1 = jnp.maximum(m1, ai)
        m2 = jnp.maximum(m2, t)
    e1 = jax.lax.bitcast_convert_type(m1, jnp.int32) & 7
    e2 = jax.lax.bitcast_convert_type(m2, jnp.int32) & 7
    s0 = jax.nn.sigmoid(m1 - m2)
    t0_ref[:, :] = s0
    t1_ref[:, :] = 1.0 - s0
    i0_ref[:, :] = e1
    i1_ref[:, :] = e2
    msk = jnp.left_shift(jnp.int32(1), e1) | jnp.left_shift(jnp.int32(1), e2)
    r = BI
    while r > 1:
        r //= 2
        msk = msk[:r, :] | msk[r:2 * r, :]

    @pl.when(ib == 0)
    def _():
        oh_ref[:, :] = msk

    @pl.when(ib > 0)
    def _():
        oh_ref[:, :] = oh_ref[:, :] | msk


def _loss_kernel(ps_ref, oh_ref, out_ref):
    ps = ps_ref[:, :]
    ohm = oh_ref[:, :]
    acc = jnp.zeros((1, 1), jnp.float32)
    for e in range(E):
        cnt = jnp.sum(((ohm >> e) & 1).astype(jnp.float32))
        acc = acc + ps[0, e] * cnt
    load_loss = E * acc * (1.0 / (B * B))
    m = jnp.sum(ps) * (1.0 / E)
    v = jnp.sum((ps - m) ** 2) * (1.0 / (E - 1))
    out_ref[:, :] = 0.01 * load_loss + 0.01 * (v / m)


def kernel(x, params, noise_raw):
    p = params
    f32 = jnp.float32
    row = lambda a: a.reshape(1, -1).astype(f32)
    bw = jnp.concatenate([p[n + "_w"] for n in ("ve", "te", "me", "re")], 0)
    bb = jnp.concatenate([p[n + "_b"] for n in ("ve", "te", "me", "re")]).reshape(1, -1)
    bg = jnp.concatenate([p[n + "_g"] for n in ("ve", "te", "me", "re")]).reshape(1, -1)
    bbeta = jnp.concatenate([p[n + "_beta"] for n in ("ve", "te", "me", "re")]).reshape(1, -1)
    gmat = jnp.kron(jnp.eye(4, dtype=f32), jnp.full((16, 16), 1.0 / 16, f32))

    nrb = B // R_BLK
    full = lambda s: pl.BlockSpec(s, lambda i: (0, 0))
    clean, ns, ps = pl.pallas_call(
        _mlp_kernel,
        grid=(nrb,),
        in_specs=[
            pl.BlockSpec((R_BLK, D), lambda i: (i, 0)),
            full((64, D)), full((1, 64)), full((1, 64)), full((1, 64)),
            full((64, 64)),
            full((32, 64)), full((1, 32)), full((4, 32)), full((1, 4)),
            full((2 * GI, GI)), full((1, 2 * GI)), full((1, 2 * GI)),
            full((1, 2 * GI)),
            full((GI, 2 * GI)), full((1, GI)), full((1, GI)), full((1, GI)),
            full((E, GI)), full((1, E)),
            full((16, 4)), full((1, 16)), full((1, 16)), full((1, 1)),
            full((E, 4)),
        ],
        out_specs=[
            pl.BlockSpec((R_BLK, E), lambda i: (i, 0)),
            pl.BlockSpec((R_BLK, 1), lambda i: (i, 0)),
            pl.BlockSpec((1, E), lambda i: (0, 0)),
        ],
        out_shape=[
            jax.ShapeDtypeStruct((B, E), f32),
            jax.ShapeDtypeStruct((B, 1), f32),
            jax.ShapeDtypeStruct((1, E), f32),
        ],
    )(x, bw, bb, bg, bbeta, gmat,
      p["rc1_w"], row(p["rc1_b"]), p["rc2_w"], row(p["rc2_b"]),
      p["g1_w"], row(p["g1_b"]), row(p["g1_g"]), row(p["g1_beta"]),
      p["g2_w"], row(p["g2_b"]), row(p["g2_g"]), row(p["g2_beta"]),
      p["g3_w"], row(p["g3_b"]),
      p["ns1_w"], row(p["ns1_b"]), p["ns2_w"], row(p["ns2_b"]),
      p["exp_spec"])

    ct = clean.T
    nt = noise_raw.T
    t0, t1, i0, i1, oh = pl.pallas_call(
        _topk_kernel,
        grid=(B // BJ, B // BI),
        in_specs=[
            pl.BlockSpec((E, BJ), lambda j, i: (0, j)),
            pl.BlockSpec((E, BJ), lambda j, i: (0, j)),
            pl.BlockSpec((BI, 1), lambda j, i: (i, 0)),
        ],
        out_specs=[
            pl.BlockSpec((BI, BJ), lambda j, i: (i, j)),
            pl.BlockSpec((BI, BJ), lambda j, i: (i, j)),
            pl.BlockSpec((BI, BJ), lambda j, i: (i, j)),
            pl.BlockSpec((BI, BJ), lambda j, i: (i, j)),
            pl.BlockSpec((1, BJ), lambda j, i: (0, j)),
        ],
        out_shape=[
            jax.ShapeDtypeStruct((B, B), f32),
            jax.ShapeDtypeStruct((B, B), f32),
            jax.ShapeDtypeStruct((B, B), jnp.int32),
            jax.ShapeDtypeStruct((B, B), jnp.int32),
            jax.ShapeDtypeStruct((1, B), jnp.int32),
        ],
    )(ct, nt, ns)

    loss = pl.pallas_call(
        _loss_kernel,
        in_specs=[pl.BlockSpec((1, E), lambda: (0, 0)),
                  pl.BlockSpec((1, B), lambda: (0, 0))],
        out_specs=pl.BlockSpec((1, 1), lambda: (0, 0)),
        out_shape=jax.ShapeDtypeStruct((1, 1), f32),
    )(ps, oh)

    tw = jnp.stack([t0, t1], axis=-1)
    idx = jnp.stack([i0, i1], axis=-1)
    return (tw, idx, loss[0, 0])


# speedup vs baseline: 1.0377x; 1.0342x over previous
import jax
import jax.numpy as jnp
from jax.experimental import pallas as pl
from jax.experimental.pallas import tpu as pltpu

B = 2048
D = 1024
E = 8
K = 2
GI = D + 64 + 4

R_BLK = 512
BI = 256
BJ = 512

_INV_SQRT2 = 0.7071067811865476


def _dot_t(a, w):
    return jax.lax.dot_general(
        a, w, (((1,), (1,)), ((), ())), precision=jax.lax.Precision.DEFAULT,
        preferred_element_type=jnp.float32)


def _gelu(v):
    return 0.5 * v * (1.0 + jax.lax.erf(v * _INV_SQRT2))


def _mlp_kernel(x_ref, bw_ref, bb_ref, bg_ref, bbeta_ref, m_ref,
                rc1w_ref, rc1b_ref, rc2w_ref, rc2b_ref,
                g1w_ref, g1b_ref, g1g_ref, g1beta_ref,
                g2w_ref, g2b_ref, g2g_ref, g2beta_ref,
                g3w_ref, g3b_ref, ns1w_ref, ns1b_ref, ns2w_ref, ns2b_ref,
                exps_ref, clean_ref, ns_ref, ps_ref):
    i = pl.program_id(0)
    xb = x_ref[:, :]
    h = jnp.maximum(_dot_t(xb, bw_ref[:, :]) + bb_ref[:, :], 0.0)
    gm = jnp.dot(h, m_ref[:, :], precision=jax.lax.Precision.HIGHEST,
                 preferred_element_type=jnp.float32)
    gx2 = jnp.dot(h * h, m_ref[:, :], precision=jax.lax.Precision.HIGHEST,
                  preferred_element_type=jnp.float32)
    gv = gx2 - gm * gm
    mf = (h - gm) * jax.lax.rsqrt(gv + 1e-5) * bg_ref[:, :] + bbeta_ref[:, :]
    rc = jnp.maximum(_dot_t(mf, rc1w_ref[:, :]) + rc1b_ref[:, :], 0.0)
    rl = _dot_t(rc, rc2w_ref[:, :]) + rc2b_ref[:, :]
    rl = rl - jnp.max(rl, axis=1, keepdims=True)
    re_ = jnp.exp(rl)
    regime = re_ / jnp.sum(re_, axis=1, keepdims=True)
    gi = jnp.concatenate([xb, mf, regime], axis=1)
    h1 = _dot_t(gi, g1w_ref[:, :]) + g1b_ref[:, :]
    m1 = jnp.sum(h1, axis=1, keepdims=True) * (1.0 / (2 * GI))
    d1 = h1 - m1
    v1 = jnp.sum(d1 * d1, axis=1, keepdims=True) * (1.0 / (2 * GI))
    h1 = _gelu(d1 * jax.lax.rsqrt(v1 + 1e-5) * g1g_ref[:, :] + g1beta_ref[:, :])
    h2 = _dot_t(h1, g2w_ref[:, :]) + g2b_ref[:, :]
    m2 = jnp.sum(h2, axis=1, keepdims=True) * (1.0 / GI)
    d2 = h2 - m2
    v2 = jnp.sum(d2 * d2, axis=1, keepdims=True) * (1.0 / GI)
    h2 = _gelu(d2 * jax.lax.rsqrt(v2 + 1e-5) * g2g_ref[:, :] + g2beta_ref[:, :])
    clean = (_dot_t(h2, g3w_ref[:, :]) + g3b_ref[:, :]
             + _dot_t(regime, exps_ref[:, :]))
    nsh = jnp.maximum(_dot_t(regime, ns1w_ref[:, :]) + ns1b_ref[:, :], 0.0)
    nsb = nsh.astype(jnp.bfloat16).astype(jnp.float32)
    nwb = ns2w_ref[:, :].astype(jnp.bfloat16).astype(jnp.float32)
    nss = jax.nn.sigmoid(
        jnp.sum(nsb * nwb, axis=1, keepdims=True) + ns2b_ref[0, 0])
    clean_ref[:, :] = clean
    ns_ref[:, :] = nss
    cm = jnp.max(clean, axis=1, keepdims=True)
    ce = jnp.exp(clean - cm)
    gs = ce / jnp.sum(ce, axis=1, keepdims=True)
    psb = jnp.sum(gs, axis=0, keepdims=True)

    @pl.when(i == 0)
    def _():
        ps_ref[:, :] = psb

    @pl.when(i > 0)
    def _():
        ps_ref[:, :] = ps_ref[:, :] + psb


def _topk_kernel(ct_ref, nt_ref, ns_ref, t0_ref, t1_ref, i0_ref, i1_ref,
                 oh_ref):
    ib = pl.program_id(1)
    nsv = ns_ref[:, :]
    neg = jnp.full((BI, BJ), -jnp.inf, jnp.float32)
    m1, m2 = neg, neg
    mask = jnp.int32(~7)
    for e in range(E):
        a = ct_ref[e:e + 1, :] + nsv * nt_ref[e:e + 1, :]
        ai = jax.lax.bitcast_convert_type(
            (jax.lax.bitcast_convert_type(a, jnp.int32) & mask) | e,
            jnp.float32)
        t = jnp.minimum(m1, ai)
        m1 = jnp.maximum(m1, ai)
        m2 = jnp.maximum(m2, t)
    e1 = jax.lax.bitcast_convert_type(m1, jnp.int32) & 7
    e2 = jax.lax.bitcast_convert_type(m2, jnp.int32) & 7
    s0 = jax.nn.sigmoid(m1 - m2)
    t0_ref[:, :] = s0
    t1_ref[:, :] = 1.0 - s0
    i0_ref[:, :] = e1
    i1_ref[:, :] = e2
    msk = jnp.left_shift(jnp.int32(1), e1) | jnp.left_shift(jnp.int32(1), e2)
    r = BI
    while r > 1:
        r //= 2
        msk = msk[:r, :] | msk[r:2 * r, :]

    @pl.when(ib == 0)
    def _():
        oh_ref[:, :] = msk

    @pl.when(ib > 0)
    def _():
        oh_ref[:, :] = oh_ref[:, :] | msk


def _loss_kernel(ps_ref, oh_ref, out_ref):
    ps = ps_ref[:, :]
    ohm = oh_ref[:, :]
    acc = jnp.zeros((1, 1), jnp.float32)
    for e in range(E):
        cnt = jnp.sum(((ohm >> e) & 1).astype(jnp.float32))
        acc = acc + ps[0, e] * cnt
    load_loss = E * acc * (1.0 / (B * B))
    m = jnp.sum(ps) * (1.0 / E)
    v = jnp.sum((ps - m) ** 2) * (1.0 / (E - 1))
    out_ref[:, :] = 0.01 * load_loss + 0.01 * (v / m)


def kernel(x, params, noise_raw):
    p = params
    f32 = jnp.float32
    row = lambda a: a.reshape(1, -1).astype(f32)
    bw = jnp.concatenate([p[n + "_w"] for n in ("ve", "te", "me", "re")], 0)
    bb = jnp.concatenate([p[n + "_b"] for n in ("ve", "te", "me", "re")]).reshape(1, -1)
    bg = jnp.concatenate([p[n + "_g"] for n in ("ve", "te", "me", "re")]).reshape(1, -1)
    bbeta = jnp.concatenate([p[n + "_beta"] for n in ("ve", "te", "me", "re")]).reshape(1, -1)
    gmat = jnp.kron(jnp.eye(4, dtype=f32), jnp.full((16, 16), 1.0 / 16, f32))

    nrb = B // R_BLK
    full = lambda s: pl.BlockSpec(s, lambda i: (0, 0))
    clean, ns, ps = pl.pallas_call(
        _mlp_kernel,
        grid=(nrb,),
        in_specs=[
            pl.BlockSpec((R_BLK, D), lambda i: (i, 0)),
            full((64, D)), full((1, 64)), full((1, 64)), full((1, 64)),
            full((64, 64)),
            full((32, 64)), full((1, 32)), full((4, 32)), full((1, 4)),
            full((2 * GI, GI)), full((1, 2 * GI)), full((1, 2 * GI)),
            full((1, 2 * GI)),
            full((GI, 2 * GI)), full((1, GI)), full((1, GI)), full((1, GI)),
            full((E, GI)), full((1, E)),
            full((16, 4)), full((1, 16)), full((1, 16)), full((1, 1)),
            full((E, 4)),
        ],
        out_specs=[
            pl.BlockSpec((R_BLK, E), lambda i: (i, 0)),
            pl.BlockSpec((R_BLK, 1), lambda i: (i, 0)),
            pl.BlockSpec((1, E), lambda i: (0, 0)),
        ],
        out_shape=[
            jax.ShapeDtypeStruct((B, E), f32),
            jax.ShapeDtypeStruct((B, 1), f32),
            jax.ShapeDtypeStruct((1, E), f32),
        ],
    )(x, bw, bb, bg, bbeta, gmat,
      p["rc1_w"], row(p["rc1_b"]), p["rc2_w"], row(p["rc2_b"]),
      p["g1_w"], row(p["g1_b"]), row(p["g1_g"]), row(p["g1_beta"]),
      p["g2_w"], row(p["g2_b"]), row(p["g2_g"]), row(p["g2_beta"]),
      p["g3_w"], row(p["g3_b"]),
      p["ns1_w"], row(p["ns1_b"]), p["ns2_w"], row(p["ns2_b"]),
      p["exp_spec"])

    ct = clean.T
    nt = noise_raw.T
    t0, t1, i0, i1, oh = pl.pallas_call(
        _topk_kernel,
        grid=(B // BJ, B // BI),
        in_specs=[
            pl.BlockSpec((E, BJ), lambda j, i: (0, j)),
            pl.BlockSpec((E, BJ), lambda j, i: (0, j)),
            pl.BlockSpec((BI, 1), lambda j, i: (i, 0)),
        ],
        out_specs=[
            pl.BlockSpec((BI, BJ), lambda j, i: (i, j)),
            pl.BlockSpec((BI, BJ), lambda j, i: (i, j)),
            pl.BlockSpec((BI, BJ), lambda j, i: (i, j)),
            pl.BlockSpec((BI, BJ), lambda j, i: (i, j)),
            pl.BlockSpec((1, BJ), lambda j, i: (0, j)),
        ],
        out_shape=[
            jax.ShapeDtypeStruct((B, B), f32),
            jax.ShapeDtypeStruct((B, B), f32),
            jax.ShapeDtypeStruct((B, B), jnp.int32),
            jax.ShapeDtypeStruct((B, B), jnp.int32),
            jax.ShapeDtypeStruct((1, B), jnp.int32),
        ],
    )(ct, nt, ns)

    loss = pl.pallas_call(
        _loss_kernel,
        in_specs=[pl.BlockSpec((1, E), lambda: (0, 0)),
                  pl.BlockSpec((1, B), lambda: (0, 0))],
        out_specs=pl.BlockSpec((1, 1), lambda: (0, 0)),
        out_shape=jax.ShapeDtypeStruct((1, 1), f32),
    )(ps, oh)

    tw = jnp.stack([t0, t1], axis=-1)
    idx = jnp.stack([i0, i1], axis=-1)
    return (tw, idx, loss[0, 0])
